# baseline (speedup 1.0000x reference)
import functools

import jax
import jax.numpy as jnp
from jax import lax
from jax.experimental import pallas as pl
from jax.experimental.pallas import tpu as pltpu
from jax.experimental.pallas import tpu_sc as plsc

_B = 4096
_D = 64
_H = 128
_NC = 2
_NS = 16
_NW = _NC * _NS
_BPW = _B // _NW
_TB = 512
_TC = 2048


def _tr_body(in_ref, o_ref):
    o_ref[...] = in_ref[...].T


def _make_transpose(v):
    grid = (v + _TC - 1) // _TC
    return pl.pallas_call(
        _tr_body,
        grid=(grid,),
        in_specs=[pl.BlockSpec((_D, _TC), lambda i: (0, i))],
        out_specs=pl.BlockSpec((_TC, _D), lambda i: (i, 0)),
        out_shape=jax.ShapeDtypeStruct((v, _D), jnp.float32),
        compiler_params=pltpu.CompilerParams(
            dimension_semantics=("arbitrary",)),
    )


def _build_sc_gather():
    mesh = plsc.VectorSubcoreMesh(core_axis_name="c", subcore_axis_name="s")

    @functools.partial(
        pl.kernel,
        mesh=mesh,
        out_type=[jax.ShapeDtypeStruct((_B, _D), jnp.float32) for _ in range(4)],
        scratch_types=(
            [pltpu.VMEM((_BPW,), jnp.int32) for _ in range(4)]
            + [pltpu.VMEM((_BPW, _D), jnp.float32) for _ in range(4)]
            + [pltpu.SemaphoreType.DMA]
        ),
    )
    def gather_k(pid, cid, mid, crid, pt, ct, mt, crt,
                 out_p, out_c, out_m, out_cr,
                 i0, i1, i2, i3, r0, r1, r2, r3, sem):
        wid = lax.axis_index("s") * _NC + lax.axis_index("c")
        base = wid * _BPW
        for src, vbuf in ((pid, i0), (cid, i1), (mid, i2), (crid, i3)):
            pltpu.sync_copy(src.at[pl.ds(base, _BPW)], vbuf)

        for idx_v, tab, rows in ((i0, pt, r0), (i1, ct, r1),
                                 (i2, mt, r2), (i3, crt, r3)):
            @pl.loop(0, _BPW // 16)
            def _(g, idx_v=idx_v, tab=tab, rows=rows):
                vec = idx_v[pl.ds(g * 16, 16)]
                for k in range(16):
                    pltpu.async_copy(tab.at[pl.ds(vec[k], 1)],
                                     rows.at[pl.ds(g * 16 + k, 1)], sem)

        for out_hbm, rows in ((out_p, r0), (out_c, r1),
                              (out_m, r2), (out_cr, r3)):
            pltpu.make_async_copy(
                out_hbm.at[pl.ds(base, _BPW)], rows, sem).wait()
            pltpu.sync_copy(rows, out_hbm.at[pl.ds(base, _BPW)])

    return gather_k


_SC_GATHER_CACHE = []


def _sc_gather():
    if not _SC_GATHER_CACHE:
        _SC_GATHER_CACHE.append(_build_sc_gather())
    return _SC_GATHER_CACHE[0]


def _mlp_body(p_ref, d_ref, c_ref, m_ref, cr_ref, w1_ref, b1_ref, w2_ref,
              b2_ref, o_ref):
    x = jnp.concatenate(
        [p_ref[...], d_ref[...], c_ref[...], m_ref[...], cr_ref[...]], axis=1)
    h = jnp.dot(x, w1_ref[...], preferred_element_type=jnp.float32)
    h = jnp.maximum(h + b1_ref[...], 0.0)
    o = jnp.dot(h, w2_ref[...], preferred_element_type=jnp.float32)
    o_ref[...] = jnp.maximum(o + b2_ref[...], 0.0)


_ROW_TILE = pl.BlockSpec((_TB, _D), lambda i: (i, 0))

_MLP = pl.pallas_call(
    _mlp_body,
    grid=(_B // _TB,),
    in_specs=[
        _ROW_TILE, _ROW_TILE, _ROW_TILE, _ROW_TILE, _ROW_TILE,
        pl.BlockSpec((5 * _D, _H), lambda i: (0, 0)),
        pl.BlockSpec((1, _H), lambda i: (0, 0)),
        pl.BlockSpec((_H, _D), lambda i: (0, 0)),
        pl.BlockSpec((1, _D), lambda i: (0, 0)),
    ],
    out_specs=_ROW_TILE,
    out_shape=jax.ShapeDtypeStruct((_B, _D), jnp.float32),
    compiler_params=pltpu.CompilerParams(
        dimension_semantics=("parallel",)),
)


def kernel(post_id, description_embedding, category_id, media_type,
           creator_id, post_table, category_table, media_table,
           creator_table, W1, b1, W2, b2):
    pid = post_id.astype(jnp.int32)
    cid = category_id.astype(jnp.int32)
    mid = media_type.astype(jnp.int32)
    crid = creator_id.astype(jnp.int32)
    pt = _make_transpose(post_table.shape[0])(post_table.T)
    crt = _make_transpose(creator_table.shape[0])(creator_table.T)
    ct = _make_transpose(category_table.shape[0])(category_table.T)
    p_e, c_e, m_e, cr_e = _sc_gather()(
        pid, cid, mid, crid, pt, ct, media_table, crt)
    return _MLP(p_e, description_embedding, c_e, m_e, cr_e,
                W1, b1.reshape(1, _H), W2, b2.reshape(1, _D))


# speedup vs baseline: 1.3253x; 1.3253x over previous
import functools

import jax
import jax.numpy as jnp
from jax import lax
from jax.experimental import pallas as pl
from jax.experimental.pallas import tpu as pltpu
from jax.experimental.pallas import tpu_sc as plsc

_B = 4096
_D = 64
_H = 128
_NC = 2
_NS = 16
_NW = _NC * _NS
_BPW = _B // _NW
_TB = 512


def _build_sc_gather():
    mesh = plsc.VectorSubcoreMesh(core_axis_name="c", subcore_axis_name="s")

    @functools.partial(
        pl.kernel,
        mesh=mesh,
        out_type=[jax.ShapeDtypeStruct((_B, _D), jnp.float32) for _ in range(4)],
        scratch_types=(
            [pltpu.VMEM((_BPW,), jnp.int32) for _ in range(4)]
            + [pltpu.VMEM((_BPW, _D), jnp.float32) for _ in range(4)]
            + [pltpu.SemaphoreType.DMA]
        ),
    )
    def gather_k(pid, cid, mid, crid, pt, ct, mt, crt,
                 out_p, out_c, out_m, out_cr,
                 i0, i1, i2, i3, r0, r1, r2, r3, sem):
        wid = lax.axis_index("s") * _NC + lax.axis_index("c")
        base = wid * _BPW
        for src, vbuf in ((pid, i0), (cid, i1), (mid, i2), (crid, i3)):
            pltpu.sync_copy(src.at[pl.ds(base, _BPW)], vbuf)

        for idx_v, tab, rows in ((i0, pt, r0), (i1, ct, r1),
                                 (i2, mt, r2), (i3, crt, r3)):
            @pl.loop(0, _BPW // 16)
            def _(g, idx_v=idx_v, tab=tab, rows=rows):
                vec = idx_v[pl.ds(g * 16, 16)]
                for k in range(16):
                    pltpu.async_copy(tab.at[pl.ds(vec[k], 1)],
                                     rows.at[pl.ds(g * 16 + k, 1)], sem)

        for out_hbm, rows in ((out_p, r0), (out_c, r1),
                              (out_m, r2), (out_cr, r3)):
            pltpu.make_async_copy(
                out_hbm.at[pl.ds(base, _BPW)], rows, sem).wait()
            pltpu.sync_copy(rows, out_hbm.at[pl.ds(base, _BPW)])

    return gather_k


_SC_GATHER_CACHE = []


def _sc_gather():
    if not _SC_GATHER_CACHE:
        _SC_GATHER_CACHE.append(_build_sc_gather())
    return _SC_GATHER_CACHE[0]


def _mlp_body(p_ref, d_ref, c_ref, m_ref, cr_ref, w1_ref, b1_ref, w2_ref,
              b2_ref, o_ref):
    x = jnp.concatenate(
        [p_ref[...], d_ref[...], c_ref[...], m_ref[...], cr_ref[...]], axis=1)
    h = jnp.dot(x, w1_ref[...], preferred_element_type=jnp.float32)
    h = jnp.maximum(h + b1_ref[...], 0.0)
    o = jnp.dot(h, w2_ref[...], preferred_element_type=jnp.float32)
    o_ref[...] = jnp.maximum(o + b2_ref[...], 0.0)


_ROW_TILE = pl.BlockSpec((_TB, _D), lambda i: (i, 0))

_MLP = pl.pallas_call(
    _mlp_body,
    grid=(_B // _TB,),
    in_specs=[
        _ROW_TILE, _ROW_TILE, _ROW_TILE, _ROW_TILE, _ROW_TILE,
        pl.BlockSpec((5 * _D, _H), lambda i: (0, 0)),
        pl.BlockSpec((1, _H), lambda i: (0, 0)),
        pl.BlockSpec((_H, _D), lambda i: (0, 0)),
        pl.BlockSpec((1, _D), lambda i: (0, 0)),
    ],
    out_specs=_ROW_TILE,
    out_shape=jax.ShapeDtypeStruct((_B, _D), jnp.float32),
    compiler_params=pltpu.CompilerParams(
        dimension_semantics=("parallel",)),
)


def kernel(post_id, description_embedding, category_id, media_type,
           creator_id, post_table, category_table, media_table,
           creator_table, W1, b1, W2, b2):
    pid = post_id.astype(jnp.int32)
    cid = category_id.astype(jnp.int32)
    mid = media_type.astype(jnp.int32)
    crid = creator_id.astype(jnp.int32)
    p_e, c_e, m_e, cr_e = _sc_gather()(
        pid, cid, mid, crid,
        post_table, category_table, media_table, creator_table)
    return _MLP(p_e, description_embedding, c_e, m_e, cr_e,
                W1, b1.reshape(1, _H), W2, b2.reshape(1, _D))


# speedup vs baseline: 1.3357x; 1.0078x over previous
import functools

import jax
import jax.numpy as jnp
from jax import lax
from jax.experimental import pallas as pl
from jax.experimental.pallas import tpu as pltpu
from jax.experimental.pallas import tpu_sc as plsc

_B = 4096
_D = 64
_H = 128
_NC = 2
_NS = 16
_NW = _NC * _NS
_BPW = _B // _NW
_TB = 512


def _build_sc_gather(n_tables):
    mesh = plsc.VectorSubcoreMesh(core_axis_name="c", subcore_axis_name="s")

    @functools.partial(
        pl.kernel,
        mesh=mesh,
        out_type=[jax.ShapeDtypeStruct((_B, _D), jnp.float32)
                  for _ in range(n_tables)],
        scratch_types=(
            [pltpu.VMEM((_BPW,), jnp.int32) for _ in range(n_tables)]
            + [pltpu.VMEM((_BPW, _D), jnp.float32) for _ in range(n_tables)]
            + [pltpu.SemaphoreType.DMA]
        ),
    )
    def gather_k(*refs):
        ids = refs[:n_tables]
        tabs = refs[n_tables:2 * n_tables]
        outs = refs[2 * n_tables:3 * n_tables]
        ibufs = refs[3 * n_tables:4 * n_tables]
        rbufs = refs[4 * n_tables:5 * n_tables]
        sem = refs[5 * n_tables]
        wid = lax.axis_index("s") * _NC + lax.axis_index("c")
        base = wid * _BPW
        for src, vbuf in zip(ids, ibufs):
            pltpu.sync_copy(src.at[pl.ds(base, _BPW)], vbuf)

        for idx_v, tab, rows in zip(ibufs, tabs, rbufs):
            @pl.loop(0, _BPW // 16)
            def _(g, idx_v=idx_v, tab=tab, rows=rows):
                vec = idx_v[pl.ds(g * 16, 16)]
                for k in range(16):
                    pltpu.async_copy(tab.at[pl.ds(vec[k], 1)],
                                     rows.at[pl.ds(g * 16 + k, 1)], sem)

        for out_hbm, rows in zip(outs, rbufs):
            pltpu.make_async_copy(
                out_hbm.at[pl.ds(base, _BPW)], rows, sem).wait()
            pltpu.sync_copy(rows, out_hbm.at[pl.ds(base, _BPW)])

    return gather_k


_SC_GATHER_CACHE = {}


def _sc_gather(n_tables):
    if n_tables not in _SC_GATHER_CACHE:
        _SC_GATHER_CACHE[n_tables] = _build_sc_gather(n_tables)
    return _SC_GATHER_CACHE[n_tables]


def _mlp_body(p_ref, d_ref, c_ref, m_ref, cr_ref, w1_ref, b1_ref, w2_ref,
              b2_ref, o_ref):
    x = jnp.concatenate(
        [p_ref[...], d_ref[...], c_ref[...], m_ref[...], cr_ref[...]], axis=1)
    h = jnp.dot(x, w1_ref[...], preferred_element_type=jnp.float32)
    h = jnp.maximum(h + b1_ref[...], 0.0)
    o = jnp.dot(h, w2_ref[...], preferred_element_type=jnp.float32)
    o_ref[...] = jnp.maximum(o + b2_ref[...], 0.0)


_ROW_TILE = pl.BlockSpec((_TB, _D), lambda i: (i, 0))

_MLP = pl.pallas_call(
    _mlp_body,
    grid=(_B // _TB,),
    in_specs=[
        _ROW_TILE, _ROW_TILE, _ROW_TILE, _ROW_TILE, _ROW_TILE,
        pl.BlockSpec((5 * _D, _H), lambda i: (0, 0)),
        pl.BlockSpec((1, _H), lambda i: (0, 0)),
        pl.BlockSpec((_H, _D), lambda i: (0, 0)),
        pl.BlockSpec((1, _D), lambda i: (0, 0)),
    ],
    out_specs=_ROW_TILE,
    out_shape=jax.ShapeDtypeStruct((_B, _D), jnp.float32),
    compiler_params=pltpu.CompilerParams(
        dimension_semantics=("parallel",)),
)


def kernel(post_id, description_embedding, category_id, media_type,
           creator_id, post_table, category_table, media_table,
           creator_table, W1, b1, W2, b2):
    pid = post_id.astype(jnp.int32)
    cid = category_id.astype(jnp.int32)
    mid = media_type.astype(jnp.int32)
    crid = creator_id.astype(jnp.int32)
    c_e, m_e, cr_e = _sc_gather(3)(
        cid, mid, crid, category_table, media_table, creator_table)
    (p_e,) = _sc_gather(1)(pid, post_table)
    return _MLP(p_e, description_embedding, c_e, m_e, cr_e,
                W1, b1.reshape(1, _H), W2, b2.reshape(1, _D))
